# Initial kernel scaffold; baseline (speedup 1.0000x reference)
import jax
import jax.numpy as jnp
from jax.experimental import pallas as pl


def kernel(x):
    raise NotImplementedError("write your pallas kernel here")


# speedup vs baseline: 4.4636x; 4.4636x over previous
import jax
import jax.numpy as jnp
from jax import lax
from jax.experimental import pallas as pl
from jax.experimental.pallas import tpu as pltpu

_T = 256


def _body(x_ref, o_ref, carry_ref):
    s = pl.program_id(1)

    @pl.when(s == 0)
    def _():
        carry_ref[...] = jnp.zeros_like(carry_ref)

    xb = x_ref[0]
    t = xb.shape[0]
    row = lax.broadcasted_iota(jnp.int32, (t, t), 0)
    col = lax.broadcasted_iota(jnp.int32, (t, t), 1)
    tri = jnp.where(row >= col, 1.0, 0.0)
    cum = jnp.dot(tri, xb, preferred_element_type=jnp.float32)
    total = cum + carry_ref[...]
    seq = lax.broadcasted_iota(jnp.int32, xb.shape, 0) + (s * t + 1)
    counts = seq.astype(jnp.float32)
    o_ref[0] = total / counts
    carry_ref[...] = total[t - 1:, :]


def kernel(x):
    B, S, F = x.shape
    T = _T
    return pl.pallas_call(
        _body,
        grid=(B, S // T),
        in_specs=[pl.BlockSpec((1, T, F), lambda b, s: (b, s, 0))],
        out_specs=pl.BlockSpec((1, T, F), lambda b, s: (b, s, 0)),
        out_shape=jax.ShapeDtypeStruct((B, S, F), x.dtype),
        scratch_shapes=[pltpu.VMEM((1, F), jnp.float32)],
        compiler_params=pltpu.CompilerParams(
            dimension_semantics=("parallel", "arbitrary"),
        ),
    )(x)


# speedup vs baseline: 5.8227x; 1.3045x over previous
import jax
import jax.numpy as jnp
from jax import lax
from jax.experimental import pallas as pl
from jax.experimental.pallas import tpu as pltpu

_T = 512


def _body(x_ref, o_ref, carry_ref):
    s = pl.program_id(1)

    @pl.when(s == 0)
    def _():
        carry_ref[...] = jnp.zeros_like(carry_ref)

    xb = x_ref[0]
    t = xb.shape[0]
    row = lax.broadcasted_iota(jnp.int32, (t, t), 0)
    col = lax.broadcasted_iota(jnp.int32, (t, t), 1)
    tri = jnp.where(row >= col, 1.0, 0.0)
    cum = jnp.dot(tri, xb, preferred_element_type=jnp.float32)
    total = cum + carry_ref[...]
    seq = lax.broadcasted_iota(jnp.int32, xb.shape, 0) + (s * t + 1)
    counts = seq.astype(jnp.float32)
    o_ref[0] = total / counts
    carry_ref[...] = total[t - 1:, :]


def kernel(x):
    B, S, F = x.shape
    T = _T
    return pl.pallas_call(
        _body,
        grid=(B, S // T),
        in_specs=[pl.BlockSpec((1, T, F), lambda b, s: (b, s, 0))],
        out_specs=pl.BlockSpec((1, T, F), lambda b, s: (b, s, 0)),
        out_shape=jax.ShapeDtypeStruct((B, S, F), x.dtype),
        scratch_shapes=[pltpu.VMEM((1, F), jnp.float32)],
        compiler_params=pltpu.CompilerParams(
            dimension_semantics=("parallel", "arbitrary"),
        ),
    )(x)


# speedup vs baseline: 6.0664x; 1.0418x over previous
import jax
import jax.numpy as jnp
from jax import lax
from jax.experimental import pallas as pl
from jax.experimental.pallas import tpu as pltpu

_T = 1024


def _body(x_ref, o_ref, carry_ref):
    s = pl.program_id(1)

    @pl.when(s == 0)
    def _():
        carry_ref[...] = jnp.zeros_like(carry_ref)

    xb = x_ref[0]
    t = xb.shape[0]
    row = lax.broadcasted_iota(jnp.int32, (t, t), 0)
    col = lax.broadcasted_iota(jnp.int32, (t, t), 1)
    tri = jnp.where(row >= col, 1.0, 0.0)
    cum = jnp.dot(tri, xb, preferred_element_type=jnp.float32)
    total = cum + carry_ref[...]
    seq = lax.broadcasted_iota(jnp.int32, xb.shape, 0) + (s * t + 1)
    counts = seq.astype(jnp.float32)
    o_ref[0] = total / counts
    carry_ref[...] = total[t - 1:, :]


def kernel(x):
    B, S, F = x.shape
    T = _T
    return pl.pallas_call(
        _body,
        grid=(B, S // T),
        in_specs=[pl.BlockSpec((1, T, F), lambda b, s: (b, s, 0))],
        out_specs=pl.BlockSpec((1, T, F), lambda b, s: (b, s, 0)),
        out_shape=jax.ShapeDtypeStruct((B, S, F), x.dtype),
        scratch_shapes=[pltpu.VMEM((1, F), jnp.float32)],
        compiler_params=pltpu.CompilerParams(
            dimension_semantics=("parallel", "arbitrary"),
        ),
    )(x)


# speedup vs baseline: 7.2135x; 1.1891x over previous
import jax
import jax.numpy as jnp
from jax import lax
from jax.experimental import pallas as pl
from jax.experimental.pallas import tpu as pltpu

_T = 1024
_C = 256


def _body(x_ref, o_ref, carry_ref):
    s = pl.program_id(1)

    @pl.when(s == 0)
    def _():
        carry_ref[...] = jnp.zeros_like(carry_ref)

    t, c = _T, _C
    xb = x_ref[0]
    row = lax.broadcasted_iota(jnp.int32, (c, c), 0)
    col = lax.broadcasted_iota(jnp.int32, (c, c), 1)
    tri = jnp.where(row >= col, 1.0, 0.0)
    iota_c = lax.broadcasted_iota(jnp.int32, (c, xb.shape[1]), 0)

    off = carry_ref[...]
    for k in range(t // c):
        chunk = xb[k * c:(k + 1) * c, :]
        cumk = jnp.dot(tri, chunk, preferred_element_type=jnp.float32)
        total = cumk + off
        counts = (iota_c + (s * t + k * c + 1)).astype(jnp.float32)
        o_ref[0, k * c:(k + 1) * c, :] = total / counts
        off = total[c - 1:, :]
    carry_ref[...] = off


def kernel(x):
    B, S, F = x.shape
    T = _T
    return pl.pallas_call(
        _body,
        grid=(B, S // T),
        in_specs=[pl.BlockSpec((1, T, F), lambda b, s: (b, s, 0))],
        out_specs=pl.BlockSpec((1, T, F), lambda b, s: (b, s, 0)),
        out_shape=jax.ShapeDtypeStruct((B, S, F), x.dtype),
        scratch_shapes=[pltpu.VMEM((1, F), jnp.float32)],
        compiler_params=pltpu.CompilerParams(
            dimension_semantics=("parallel", "arbitrary"),
        ),
    )(x)


# speedup vs baseline: 7.3967x; 1.0254x over previous
import jax
import jax.numpy as jnp
from jax import lax
from jax.experimental import pallas as pl
from jax.experimental.pallas import tpu as pltpu

_T = 2048
_C = 256


def _body(x_ref, o_ref, carry_ref):
    s = pl.program_id(1)

    @pl.when(s == 0)
    def _():
        carry_ref[...] = jnp.zeros_like(carry_ref)

    t, c = _T, _C
    xb = x_ref[0]
    row = lax.broadcasted_iota(jnp.int32, (c, c), 0)
    col = lax.broadcasted_iota(jnp.int32, (c, c), 1)
    tri = jnp.where(row >= col, 1.0, 0.0)
    iota_c = lax.broadcasted_iota(jnp.int32, (c, xb.shape[1]), 0)

    off = carry_ref[...]
    for k in range(t // c):
        chunk = xb[k * c:(k + 1) * c, :]
        cumk = jnp.dot(tri, chunk, preferred_element_type=jnp.float32)
        total = cumk + off
        counts = (iota_c + (s * t + k * c + 1)).astype(jnp.float32)
        o_ref[0, k * c:(k + 1) * c, :] = total / counts
        off = total[c - 1:, :]
    carry_ref[...] = off


def kernel(x):
    B, S, F = x.shape
    T = _T
    return pl.pallas_call(
        _body,
        grid=(B, S // T),
        in_specs=[pl.BlockSpec((1, T, F), lambda b, s: (b, s, 0))],
        out_specs=pl.BlockSpec((1, T, F), lambda b, s: (b, s, 0)),
        out_shape=jax.ShapeDtypeStruct((B, S, F), x.dtype),
        scratch_shapes=[pltpu.VMEM((1, F), jnp.float32)],
        compiler_params=pltpu.CompilerParams(
            dimension_semantics=("parallel", "arbitrary"),
        ),
    )(x)
